# baseline (speedup 1.0000x reference)
import functools
import math

import jax
import jax.numpy as jnp
from jax import lax
from jax.experimental import pallas as pl
from jax.experimental.pallas import tpu as pltpu
from jax.experimental.pallas import tpu_sc as plsc

B, N, M, D_FEAT = 32, 4096, 32, 3
NPAR = 8
LANES = 16
MH = M // 2
RC = 12.0
ETA = 0.04
RS_STEP = 1.5
G_COEF = 2.0 * ETA * RS_STEP
C_K = [math.exp(-ETA * (RS_STEP * k) ** 2) for k in range(NPAR)]
S0, S1, S2, S3 = 0.999999242, -0.166656797, 8.31322559e-03, -1.85234488e-04


def _sc_layer(xt, nbp, zw):
    mesh = plsc.VectorSubcoreMesh(core_axis_name="c", subcore_axis_name="s")

    @functools.partial(
        pl.kernel,
        mesh=mesh,
        compiler_params=pltpu.CompilerParams(
            needs_layout_passes=False, use_tc_tiling_on_sc=False),
        out_type=jax.ShapeDtypeStruct((B, N * NPAR), jnp.float32),
        scratch_types=[
            pltpu.VMEM((N,), jnp.float32),
            pltpu.VMEM((N,), jnp.float32),
            pltpu.VMEM((N,), jnp.float32),
            pltpu.VMEM((N * MH,), jnp.int32),
            pltpu.VMEM((N,), jnp.int32),
            pltpu.VMEM((N * NPAR,), jnp.float32),
        ],
    )
    def k(xt_hbm, nb_hbm, zw_hbm, out_hbm, xp, yp, zp, nb_v, zw_v, stage):
        b = lax.axis_index("s") * 2 + lax.axis_index("c")
        pltpu.sync_copy(xt_hbm.at[3 * b + 0], xp)
        pltpu.sync_copy(xt_hbm.at[3 * b + 1], yp)
        pltpu.sync_copy(xt_hbm.at[3 * b + 2], zp)
        pltpu.sync_copy(nb_hbm.at[b], nb_v)
        pltpu.sync_copy(zw_hbm.at[b], zw_v)

        lanes = lax.iota(jnp.int32, LANES)
        lane_w = lanes * MH
        lane_p = lanes * NPAR

        def group_body(g, carry):
            a0 = g * LANES
            xo = xp[pl.ds(a0, LANES)]
            yo = yp[pl.ds(a0, LANES)]
            zo = zp[pl.ds(a0, LANES)]
            wbits = zw_v[pl.ds(a0, LANES)]
            gbase = g * (LANES * MH)

            def pair(ni, mbit, accs):
                xn = plsc.load_gather(xp, [ni])
                yn = plsc.load_gather(yp, [ni])
                zn = plsc.load_gather(zp, [ni])
                dx = xn - xo
                dy = yn - yo
                dz = zn - zo
                s = dx * dx + dy * dy + dz * dz
                i = lax.bitcast_convert_type(s, jnp.int32)
                i = 0x5F3759DF - lax.shift_right_arithmetic(i, 1)
                y = lax.bitcast_convert_type(i, jnp.float32)
                hs = 0.5 * s
                y = y * (1.5 - hs * y * y)
                y = y * (1.5 - hs * y * y)
                r = s * y
                rcl = jnp.minimum(r, RC)
                u = rcl * (math.pi / RC) - (0.5 * math.pi)
                u2 = u * u
                p = S0 + u2 * (S1 + u2 * (S2 + u2 * S3))
                fc = 0.5 - (0.5 * u) * p
                a = jnp.exp(-ETA * (rcl * rcl))
                gg = jnp.exp(G_COEF * rcl)
                t = (fc * a) * mbit.astype(jnp.float32)
                new = []
                for kk in range(NPAR):
                    new.append(accs[kk] + t)
                    if kk < NPAR - 1:
                        t = t * gg
                return tuple(new)

            accs = (jnp.zeros((LANES,), jnp.float32),) * NPAR
            for mm in range(MH):
                w = plsc.load_gather(nb_v, [lane_w + (gbase + mm)])
                ni0 = jnp.bitwise_and(w, 0xFFFF)
                ni1 = lax.shift_right_logical(w, 16)
                bit0 = jnp.bitwise_and(
                    lax.shift_right_logical(wbits, 2 * mm), 1)
                bit1 = jnp.bitwise_and(
                    lax.shift_right_logical(wbits, 2 * mm + 1), 1)
                accs = pair(ni0, bit0, accs)
                accs = pair(ni1, bit1, accs)

            sbase = g * (LANES * NPAR)
            for kk in range(NPAR):
                plsc.store_scatter(
                    stage, [lane_p + (sbase + kk)], accs[kk] * C_K[kk])
            return carry

        lax.fori_loop(0, N // LANES, group_body, 0)
        pltpu.sync_copy(stage, out_hbm.at[b])

    return k(xt, nbp, zw)


def _bn(layer):
    cols = N * NPAR // 16

    def body(x_ref, o_ref):
        x = x_ref[...]
        mu = jnp.mean(x, axis=0, keepdims=True)
        d = x - mu
        var = jnp.mean(d * d, axis=0, keepdims=True)
        o_ref[...] = d * lax.rsqrt(var + 0.001)

    return pl.pallas_call(
        body,
        grid=(16,),
        in_specs=[pl.BlockSpec((B, cols), lambda i: (0, i))],
        out_specs=pl.BlockSpec((B, cols), lambda i: (0, i)),
        out_shape=jax.ShapeDtypeStruct((B, N * NPAR), jnp.float32),
    )(layer)


def kernel(X, Nbrs, Nbrs_Z):
    xt = jnp.transpose(X, (0, 2, 1)).reshape(B * D_FEAT, N)
    nb16 = Nbrs.astype(jnp.int16).reshape(B, N * MH, 2)
    nbp = lax.bitcast_convert_type(nb16, jnp.int32)
    zbits = lax.shift_left(
        (Nbrs_Z != 0).astype(jnp.int32),
        jnp.arange(M, dtype=jnp.int32)[None, None, :])
    zw = jnp.sum(zbits, axis=-1).astype(jnp.int32)
    layer = _sc_layer(xt, nbp, zw)
    out = _bn(layer)
    return out.reshape(B, N, NPAR)


# speedup vs baseline: 2.4048x; 2.4048x over previous
import functools
import math

import jax
import jax.numpy as jnp
from jax import lax
from jax.experimental import pallas as pl
from jax.experimental.pallas import tpu as pltpu
from jax.experimental.pallas import tpu_sc as plsc

B, N, M, D_FEAT = 32, 4096, 32, 3
NPAR = 8
LANES = 16
MH = M // 2
RC = 12.0
ETA = 0.04
RS_STEP = 1.5
G_COEF = 2.0 * ETA * RS_STEP
C_K = [math.exp(-ETA * (RS_STEP * k) ** 2) for k in range(NPAR)]
S0, S1, S2, S3 = 0.999999242, -0.166656797, 8.31322559e-03, -1.85234488e-04


def _sc_layer(xt, nbp, zw):
    mesh = plsc.VectorSubcoreMesh(core_axis_name="c", subcore_axis_name="s")

    @functools.partial(
        pl.kernel,
        mesh=mesh,
        compiler_params=pltpu.CompilerParams(
            needs_layout_passes=False, use_tc_tiling_on_sc=False),
        out_type=jax.ShapeDtypeStruct((B, N * NPAR), jnp.float32),
        scratch_types=[
            pltpu.VMEM((N,), jnp.float32),
            pltpu.VMEM((N,), jnp.float32),
            pltpu.VMEM((N,), jnp.float32),
            pltpu.VMEM((N * MH,), jnp.int32),
            pltpu.VMEM((N,), jnp.int32),
            pltpu.VMEM((N * NPAR,), jnp.float32),
        ],
    )
    def k(xt_hbm, nb_hbm, zw_hbm, out_hbm, xp, yp, zp, nb_v, zw_v, stage):
        b = lax.axis_index("s") * 2 + lax.axis_index("c")
        pltpu.sync_copy(xt_hbm.at[3 * b + 0], xp)
        pltpu.sync_copy(xt_hbm.at[3 * b + 1], yp)
        pltpu.sync_copy(xt_hbm.at[3 * b + 2], zp)
        pltpu.sync_copy(nb_hbm.at[b], nb_v)
        pltpu.sync_copy(zw_hbm.at[b], zw_v)

        lanes = lax.iota(jnp.int32, LANES)
        lane_w = lanes * MH
        lane_p = lanes * NPAR

        def group_body(g, carry):
            a0 = g * LANES
            xo = xp[pl.ds(a0, LANES)]
            yo = yp[pl.ds(a0, LANES)]
            zo = zp[pl.ds(a0, LANES)]
            wbits = zw_v[pl.ds(a0, LANES)]
            gbase = g * (LANES * MH)

            def pair(ni, mbit, accs):
                xn = plsc.load_gather(xp, [ni])
                yn = plsc.load_gather(yp, [ni])
                zn = plsc.load_gather(zp, [ni])
                dx = xn - xo
                dy = yn - yo
                dz = zn - zo
                s = dx * dx + dy * dy + dz * dz
                i = lax.bitcast_convert_type(s, jnp.int32)
                i = 0x5F3759DF - lax.shift_right_arithmetic(i, 1)
                y = lax.bitcast_convert_type(i, jnp.float32)
                hs = 0.5 * s
                y = y * (1.5 - hs * y * y)
                y = y * (1.5 - hs * y * y)
                r = s * y
                rcl = jnp.minimum(r, RC)
                u = rcl * (math.pi / RC) - (0.5 * math.pi)
                u2 = u * u
                p = S0 + u2 * (S1 + u2 * (S2 + u2 * S3))
                fc = 0.5 - (0.5 * u) * p
                a = jnp.exp(-ETA * (rcl * rcl))
                gg = jnp.exp(G_COEF * rcl)
                t = (fc * a) * mbit.astype(jnp.float32)
                new = []
                for kk in range(NPAR):
                    new.append(accs[kk] + t)
                    if kk < NPAR - 1:
                        t = t * gg
                return tuple(new)

            accs = (jnp.zeros((LANES,), jnp.float32),) * NPAR
            for mm in range(MH):
                w = plsc.load_gather(nb_v, [lane_w + (gbase + mm)])
                ni0 = jnp.bitwise_and(w, 0xFFFF)
                ni1 = lax.shift_right_logical(w, 16)
                bit0 = jnp.bitwise_and(
                    lax.shift_right_logical(wbits, mm), 1)
                bit1 = jnp.bitwise_and(
                    lax.shift_right_logical(wbits, mm + MH), 1)
                accs = pair(ni0, bit0, accs)
                accs = pair(ni1, bit1, accs)

            sbase = g * (LANES * NPAR)
            for kk in range(NPAR):
                plsc.store_scatter(
                    stage, [lane_p + (sbase + kk)], accs[kk] * C_K[kk])
            return carry

        lax.fori_loop(0, N // LANES, group_body, 0)
        pltpu.sync_copy(stage, out_hbm.at[b])

    return k(xt, nbp, zw)


def _bn(layer):
    cols = N * NPAR // 16

    def body(x_ref, o_ref):
        x = x_ref[...]
        mu = jnp.mean(x, axis=0, keepdims=True)
        d = x - mu
        var = jnp.mean(d * d, axis=0, keepdims=True)
        o_ref[...] = d * lax.rsqrt(var + 0.001)

    return pl.pallas_call(
        body,
        grid=(16,),
        in_specs=[pl.BlockSpec((B, cols), lambda i: (0, i))],
        out_specs=pl.BlockSpec((B, cols), lambda i: (0, i)),
        out_shape=jax.ShapeDtypeStruct((B, N * NPAR), jnp.float32),
    )(layer)


def kernel(X, Nbrs, Nbrs_Z):
    xt = jnp.transpose(X, (0, 2, 1)).reshape(B * D_FEAT, N)
    nbp = jnp.bitwise_or(
        Nbrs[:, :, :MH],
        lax.shift_left(Nbrs[:, :, MH:], 16)).reshape(B, N * MH)
    zbits = lax.shift_left(
        (Nbrs_Z != 0).astype(jnp.int32),
        jnp.arange(M, dtype=jnp.int32)[None, None, :])
    zw = jnp.sum(zbits, axis=-1).astype(jnp.int32)
    layer = _sc_layer(xt, nbp, zw)
    out = _bn(layer)
    return out.reshape(B, N, NPAR)
